# baseline (speedup 1.0000x reference)
import jax
import jax.numpy as jnp
from jax import lax
from jax.experimental import pallas as pl
from jax.experimental.pallas import tpu as pltpu
from jax.experimental.pallas import tpu_sc as plsc

_NC, _NS, _L = 2, 16, 16
_NW = _NC * _NS
_CHUNK = 32
_NBUF = 3
_PF = 2
_SC_ROWS = 6144
_TC_BLOCK = 1024


def _sc_body(tid_hbm, emb_hbm, tok_hbm, out_hbm, idx_v, row_v, buf, *sems):
    wid = lax.axis_index("s") * _NC + lax.axis_index("c")
    sc_rows, d_model = out_hbm.shape
    rows_per_w = sc_rows // _NW
    base = wid * rows_per_w
    nchunks = rows_per_w // _CHUNK

    pltpu.sync_copy(tid_hbm, idx_v)
    pltpu.async_copy(emb_hbm.at[idx_v], row_v, sems[0]).wait()

    def bufslice(b):
        return buf.at[pl.ds(b * _CHUNK, _CHUNK)]

    def tokslice(c):
        return tok_hbm.at[pl.ds(base + c * _CHUNK, _CHUNK)]

    def outslice(c):
        return out_hbm.at[pl.ds(base + c * _CHUNK, _CHUNK)]

    def compute(b):
        @pl.loop(0, d_model // _L)
        def _d(d):
            col = d * _L
            rv = row_v[0, pl.ds(col, _L)]

            @plsc.parallel_loop(0, _CHUNK, unroll=8)
            def _r(r):
                buf[b * _CHUNK + r, pl.ds(col, _L)] += rv

    for c in range(_PF):
        pltpu.async_copy(tokslice(c), bufslice(c % _NBUF), sems[c % _NBUF])

    for c in range(nchunks):
        b = c % _NBUF
        pltpu.make_async_copy(tokslice(c), bufslice(b), sems[b]).wait()
        compute(b)
        pltpu.async_copy(bufslice(b), outslice(c), sems[b])
        pc = c + _PF
        if pc < nchunks:
            pb = pc % _NBUF
            if pc >= _NBUF:
                pltpu.make_async_copy(
                    bufslice(pb), outslice(pc - _NBUF), sems[pb]).wait()
            pltpu.async_copy(tokslice(pc), bufslice(pb), sems[pb])

    for c in range(nchunks - _NBUF, nchunks):
        b = c % _NBUF
        pltpu.make_async_copy(bufslice(b), outslice(c), sems[b]).wait()


def _tc_body(id_ref, emb_ref, tok_ref, out_ref):
    idx = id_ref[0]
    row = emb_ref[pl.ds(idx, 1), :]
    out_ref[...] = tok_ref[...] + row


def kernel(tokens, embed_weight, type_id):
    B, N, D = tokens.shape
    rows = B * N
    flat = tokens.reshape(rows, D)
    tid_vec = jnp.full((8,), type_id, jnp.int32)

    mesh = plsc.VectorSubcoreMesh(
        core_axis_name="c", subcore_axis_name="s",
        num_cores=_NC, num_subcores=_NS)
    sc = pl.kernel(
        _sc_body,
        out_type=jax.ShapeDtypeStruct((_SC_ROWS, D), tokens.dtype),
        mesh=mesh,
        scratch_types=[
            pltpu.VMEM((8,), jnp.int32),
            pltpu.VMEM((8, D), jnp.float32),
            pltpu.VMEM((_NBUF * _CHUNK, D), jnp.float32),
        ] + [pltpu.SemaphoreType.DMA] * _NBUF,
    )
    sc_out = sc(tid_vec, embed_weight, flat)

    tc_rows = rows - _SC_ROWS
    base_blk = _SC_ROWS // _TC_BLOCK
    tid_s = jnp.asarray(type_id, jnp.int32).reshape(1)
    tc_out = pl.pallas_call(
        _tc_body,
        grid=(tc_rows // _TC_BLOCK,),
        in_specs=[
            pl.BlockSpec(memory_space=pltpu.SMEM),
            pl.BlockSpec(embed_weight.shape, lambda i: (0, 0)),
            pl.BlockSpec((_TC_BLOCK, D), lambda i: (base_blk + i, 0)),
        ],
        out_specs=pl.BlockSpec((_TC_BLOCK, D), lambda i: (i, 0)),
        out_shape=jax.ShapeDtypeStruct((tc_rows, D), tokens.dtype),
    )(tid_s, embed_weight, flat)

    return jnp.concatenate([sc_out, tc_out], axis=0).reshape(B, N, D)


# speedup vs baseline: 1.4299x; 1.4299x over previous
import jax
import jax.numpy as jnp
from jax import lax
from jax.experimental import pallas as pl
from jax.experimental.pallas import tpu as pltpu
from jax.experimental.pallas import tpu_sc as plsc

_NC, _NS, _L = 2, 16, 16
_NW = _NC * _NS
_CHUNK = 32
_NBUF = 3
_PF = 2


def _sc_body(tid_hbm, emb_hbm, tok_hbm, out_hbm, idx_v, row_v, buf, *sems):
    wid = lax.axis_index("s") * _NC + lax.axis_index("c")
    rows, d_model = tok_hbm.shape
    rows_per_w = rows // _NW
    base = wid * rows_per_w
    nchunks = rows_per_w // _CHUNK

    pltpu.sync_copy(tid_hbm, idx_v)
    pltpu.async_copy(emb_hbm.at[idx_v], row_v, sems[0]).wait()

    def bufslice(b):
        return buf.at[pl.ds(b * _CHUNK, _CHUNK)]

    def tokslice(c):
        return tok_hbm.at[pl.ds(base + c * _CHUNK, _CHUNK)]

    def outslice(c):
        return out_hbm.at[pl.ds(base + c * _CHUNK, _CHUNK)]

    def compute(b):
        @pl.loop(0, d_model // _L)
        def _d(d):
            col = d * _L
            rv = row_v[0, pl.ds(col, _L)]

            @plsc.parallel_loop(0, _CHUNK, unroll=8)
            def _r(r):
                buf[b * _CHUNK + r, pl.ds(col, _L)] += rv

    for c in range(_PF):
        pltpu.async_copy(tokslice(c), bufslice(c % _NBUF), sems[c % _NBUF])

    for c in range(nchunks):
        b = c % _NBUF
        pltpu.make_async_copy(tokslice(c), bufslice(b), sems[b]).wait()
        compute(b)
        pltpu.async_copy(bufslice(b), outslice(c), sems[b])
        pc = c + _PF
        if pc < nchunks:
            pb = pc % _NBUF
            if pc >= _NBUF:
                pltpu.make_async_copy(
                    bufslice(pb), outslice(pc - _NBUF), sems[pb]).wait()
            pltpu.async_copy(tokslice(pc), bufslice(pb), sems[pb])

    for c in range(nchunks - _NBUF, nchunks):
        b = c % _NBUF
        pltpu.make_async_copy(bufslice(b), outslice(c), sems[b]).wait()


def kernel(tokens, embed_weight, type_id):
    B, N, D = tokens.shape
    rows = B * N
    flat = tokens.reshape(rows, D)
    tid_vec = jnp.full((8,), type_id, jnp.int32)

    mesh = plsc.VectorSubcoreMesh(
        core_axis_name="c", subcore_axis_name="s",
        num_cores=_NC, num_subcores=_NS)
    sc = pl.kernel(
        _sc_body,
        out_type=jax.ShapeDtypeStruct((rows, D), tokens.dtype),
        mesh=mesh,
        scratch_types=[
            pltpu.VMEM((8,), jnp.int32),
            pltpu.VMEM((8, D), jnp.float32),
            pltpu.VMEM((_NBUF * _CHUNK, D), jnp.float32),
        ] + [pltpu.SemaphoreType.DMA] * _NBUF,
    )
    out = sc(tid_vec, embed_weight, flat)
    return out.reshape(B, N, D)
